# baseline (speedup 1.0000x reference)
import functools

import jax
import jax.numpy as jnp
from jax import lax
from jax.experimental import pallas as pl
from jax.experimental.pallas import tpu as pltpu
from jax.experimental.pallas import tpu_sc as plsc

N_NODES = 10000
FDIM = 128
NK = 10
JA = 1.0
JB = 1.0

NCHUNK = 80
EDGES_PER_TILE = NCHUNK * 128
E_PAD = 32 * EDGES_PER_TILE
N_ACC = 10240
ROWS_PER_TILE = N_ACC // 16

_sc_mesh = plsc.VectorSubcoreMesh(core_axis_name="c", subcore_axis_name="s")


HALF = NCHUNK // 2


@functools.partial(
    pl.kernel,
    mesh=_sc_mesh,
    out_type=jax.ShapeDtypeStruct((2, N_ACC, FDIM), jnp.float32),
    scratch_types=[
        pltpu.VMEM((HALF, 128), jnp.int32),
        pltpu.VMEM((HALF, 128), jnp.int32),
        pltpu.VMEM((2, 128, FDIM), jnp.float32),
        pltpu.VMEM_SHARED((N_ACC, FDIM), jnp.float32),
        pltpu.SemaphoreType.DMA,
        pltpu.SemaphoreType.DMA,
        pltpu.SemaphoreType.DMA,
        pltpu.SemaphoreType.DMA,
    ],
)
def _sc_spmm(col_hbm, row_hbm, ht_hbm, zeros_hbm, p_hbm,
             col_v, row_v, ring, acc, g0, g1, s0, s1):
    c = lax.axis_index("c")
    s = lax.axis_index("s")
    tile = c * 16 + s
    gsem = (g0, g1)
    ssem = (s0, s1)
    pltpu.sync_copy(zeros_hbm.at[pl.ds(s * ROWS_PER_TILE, ROWS_PER_TILE)],
                    acc.at[pl.ds(s * ROWS_PER_TILE, ROWS_PER_TILE)])
    plsc.subcore_barrier()

    for hf in range(2):
        base = tile * NCHUNK + hf * HALF
        pltpu.sync_copy(col_hbm.at[pl.ds(base, HALF)], col_v)
        pltpu.sync_copy(row_hbm.at[pl.ds(base, HALF)], row_v)
        pltpu.async_copy(ht_hbm.at[col_v.at[0, pl.ds(0, 64)]],
                         ring.at[0, pl.ds(0, 64)], gsem[0])
        pltpu.async_copy(ht_hbm.at[col_v.at[0, pl.ds(64, 64)]],
                         ring.at[0, pl.ds(64, 64)], gsem[0])
        for j in range(HALF):
            b = j % 2
            pltpu.make_async_copy(ht_hbm.at[col_v.at[j, pl.ds(0, 64)]],
                                  ring.at[b, pl.ds(0, 64)], gsem[b]).wait()
            pltpu.make_async_copy(ht_hbm.at[col_v.at[j, pl.ds(64, 64)]],
                                  ring.at[b, pl.ds(64, 64)], gsem[b]).wait()
            if j >= 1:
                pltpu.make_async_copy(ring.at[1 - b], acc.at[row_v.at[0]],
                                      ssem[1 - b]).wait()
            if j + 1 < HALF:
                pltpu.async_copy(ht_hbm.at[col_v.at[j + 1, pl.ds(0, 64)]],
                                 ring.at[1 - b, pl.ds(0, 64)], gsem[1 - b])
                pltpu.async_copy(ht_hbm.at[col_v.at[j + 1, pl.ds(64, 64)]],
                                 ring.at[1 - b, pl.ds(64, 64)], gsem[1 - b])
            pltpu.async_copy(ring.at[b], acc.at[row_v.at[j]], ssem[b], add=True)
        pltpu.make_async_copy(ring.at[(HALF - 1) % 2], acc.at[row_v.at[0]],
                              ssem[(HALF - 1) % 2]).wait()
    plsc.subcore_barrier()
    pltpu.sync_copy(acc.at[pl.ds(s * ROWS_PER_TILE, ROWS_PER_TILE)],
                    p_hbm.at[c].at[pl.ds(s * ROWS_PER_TILE, ROWS_PER_TILE)])


@functools.partial(
    pl.kernel,
    mesh=_sc_mesh,
    out_type=jax.ShapeDtypeStruct((2, N_ACC, FDIM), jnp.float32),
    scratch_types=[
        pltpu.VMEM((NCHUNK, 128), jnp.int32),
        pltpu.VMEM((128, FDIM), jnp.float32),
        pltpu.VMEM_SHARED((N_ACC, FDIM), jnp.float32),
        pltpu.SemaphoreType.DMA,
    ],
)
def _sc_degree(row_hbm, ones_hbm, zeros_hbm, p_hbm, row_v, ones_v, acc, sem):
    c = lax.axis_index("c")
    s = lax.axis_index("s")
    tile = c * 16 + s
    pltpu.sync_copy(row_hbm.at[pl.ds(tile * NCHUNK, NCHUNK)], row_v)
    pltpu.sync_copy(ones_hbm, ones_v)
    pltpu.sync_copy(zeros_hbm.at[pl.ds(s * ROWS_PER_TILE, ROWS_PER_TILE)],
                    acc.at[pl.ds(s * ROWS_PER_TILE, ROWS_PER_TILE)])
    plsc.subcore_barrier()
    for grp in range(NCHUNK // 16):
        for j in range(16):
            pltpu.async_copy(ones_v, acc.at[row_v.at[grp * 16 + j]], sem,
                             add=True)
        for j in range(16):
            pltpu.make_async_copy(ones_v, acc.at[row_v.at[0]], sem).wait()
    plsc.subcore_barrier()
    pltpu.sync_copy(acc.at[pl.ds(s * ROWS_PER_TILE, ROWS_PER_TILE)],
                    p_hbm.at[c].at[pl.ds(s * ROWS_PER_TILE, ROWS_PER_TILE)])


BLK = 400


def _mlp_body(x_ref, w_ref, b_ref, deg_ref, wf_ref, bf_ref,
              h_ref, ht_ref, hk_ref, q_ref):
    i = pl.program_id(0)
    h = jnp.maximum(jnp.dot(x_ref[...], w_ref[...],
                            preferred_element_type=jnp.float32) + b_ref[...], 0.0)
    deg = deg_ref[0, :, 0:1] + deg_ref[1, :, 0:1] + 1.0
    dis = lax.rsqrt(deg)
    h_ref[...] = h
    ht_ref[...] = h * dis
    hk = jnp.dot(h, wf_ref[...],
                 preferred_element_type=jnp.float32) + bf_ref[...]
    hk_ref[...] = hk
    part = jnp.sum(hk, axis=0, keepdims=True)

    @pl.when(i == 0)
    def _():
        q_ref[...] = part

    @pl.when(i != 0)
    def _():
        q_ref[...] += part


def _tc_mlp(x, w, b, degs, wf0, bf0):
    return pl.pallas_call(
        _mlp_body,
        grid=(N_NODES // BLK,),
        in_specs=[
            pl.BlockSpec((BLK, FDIM), lambda i: (i, 0)),
            pl.BlockSpec((FDIM, FDIM), lambda i: (0, 0)),
            pl.BlockSpec((FDIM,), lambda i: (0,)),
            pl.BlockSpec((2, BLK, 8), lambda i: (0, i, 0)),
            pl.BlockSpec((FDIM, FDIM), lambda i: (0, 0)),
            pl.BlockSpec((1, FDIM), lambda i: (0, 0)),
        ],
        out_specs=[
            pl.BlockSpec((BLK, FDIM), lambda i: (i, 0)),
            pl.BlockSpec((BLK, FDIM), lambda i: (i, 0)),
            pl.BlockSpec((BLK, FDIM), lambda i: (i, 0)),
            pl.BlockSpec((1, FDIM), lambda i: (0, 0)),
        ],
        out_shape=[
            jax.ShapeDtypeStruct((N_NODES, FDIM), jnp.float32),
            jax.ShapeDtypeStruct((N_NODES, FDIM), jnp.float32),
            jax.ShapeDtypeStruct((N_NODES, FDIM), jnp.float32),
            jax.ShapeDtypeStruct((1, FDIM), jnp.float32),
        ],
    )(x, w, b, degs, wf0, bf0)


def _combine_body(p_ref, z_ref, zp_ref, deg_ref, wf_ref, bf_ref,
                  zn_ref, htn_ref, hk_ref, q_ref, *, c0, c1, c2):
    i = pl.program_id(0)
    deg = deg_ref[0, :, 0:1] + deg_ref[1, :, 0:1] + 1.0
    dis = lax.rsqrt(deg)
    z = z_ref[...]
    sz = dis * (p_ref[0] + p_ref[1]) + (dis * dis) * z
    zn = c0 * sz + c1 * z + c2 * zp_ref[...]
    zn_ref[...] = zn
    htn_ref[...] = zn * dis
    hk = jnp.dot(zn, wf_ref[...],
                 preferred_element_type=jnp.float32) + bf_ref[...]
    hk_ref[...] = hk
    part = jnp.sum(hk, axis=0, keepdims=True)

    @pl.when(i == 0)
    def _():
        q_ref[...] = part

    @pl.when(i != 0)
    def _():
        q_ref[...] += part


def _tc_combine(p, z, zp, degs, wf, bf, c0, c1, c2):
    body = functools.partial(_combine_body, c0=c0, c1=c1, c2=c2)
    return pl.pallas_call(
        body,
        grid=(N_NODES // BLK,),
        in_specs=[
            pl.BlockSpec((2, BLK, FDIM), lambda i: (0, i, 0)),
            pl.BlockSpec((BLK, FDIM), lambda i: (i, 0)),
            pl.BlockSpec((BLK, FDIM), lambda i: (i, 0)),
            pl.BlockSpec((2, BLK, 8), lambda i: (0, i, 0)),
            pl.BlockSpec((FDIM, FDIM), lambda i: (0, 0)),
            pl.BlockSpec((1, FDIM), lambda i: (0, 0)),
        ],
        out_specs=[
            pl.BlockSpec((BLK, FDIM), lambda i: (i, 0)),
            pl.BlockSpec((BLK, FDIM), lambda i: (i, 0)),
            pl.BlockSpec((BLK, FDIM), lambda i: (i, 0)),
            pl.BlockSpec((1, FDIM), lambda i: (0, 0)),
        ],
        out_shape=[
            jax.ShapeDtypeStruct((N_NODES, FDIM), jnp.float32),
            jax.ShapeDtypeStruct((N_NODES, FDIM), jnp.float32),
            jax.ShapeDtypeStruct((N_NODES, FDIM), jnp.float32),
            jax.ShapeDtypeStruct((1, FDIM), jnp.float32),
        ],
    )(p, z, zp, degs, wf, bf)


def _attn_body(*refs):
    h_refs = refs[:NK + 1]
    qsum_ref, wc_ref, bc_ref, out_ref, zt_ref = refs[NK + 1:]
    q = qsum_ref[...] * (1.0 / N_NODES)
    cols = []
    for k in range(NK + 1):
        hk = h_refs[k][...]
        sk = jnp.sum(hk * q[k][None, :], axis=1, keepdims=True)
        cols.append(jnp.tanh(sk))
    scores = jnp.concatenate(cols, axis=1)
    m = jnp.max(scores, axis=1, keepdims=True)
    e = jnp.exp(scores - m)
    alpha = e / jnp.sum(e, axis=1, keepdims=True)
    zt = jnp.zeros_like(h_refs[0][...])
    for k in range(NK + 1):
        zt = zt + alpha[:, k:k + 1] * h_refs[k][...]
    zt = jnp.maximum(zt, 0.0)
    zt_ref[...] = zt
    out_ref[...] = jnp.dot(zt, wc_ref[...],
                           preferred_element_type=jnp.float32) + bc_ref[...]


def _tc_attn(hs, qsum, w_cls, b_cls, nout):
    hspec = pl.BlockSpec((BLK, FDIM), lambda i: (i, 0))
    return pl.pallas_call(
        _attn_body,
        grid=(N_NODES // BLK,),
        in_specs=[hspec] * (NK + 1) + [
            pl.BlockSpec((NK + 1, FDIM), lambda i: (0, 0)),
            pl.BlockSpec((FDIM, nout), lambda i: (0, 0)),
            pl.BlockSpec((nout,), lambda i: (0,)),
        ],
        out_specs=[
            pl.BlockSpec((BLK, nout), lambda i: (i, 0)),
            pl.BlockSpec((BLK, FDIM), lambda i: (i, 0)),
        ],
        out_shape=[
            jax.ShapeDtypeStruct((N_NODES, nout), jnp.float32),
            jax.ShapeDtypeStruct((N_NODES, FDIM), jnp.float32),
        ],
    )(*hs, qsum, w_cls, b_cls)


def _jacobi_coefs(k_idx):
    a, b = JA, JB
    if k_idx == 1:
        return (a + b + 2.0) / 2.0, (a - b) / 2.0, 0.0
    k = float(k_idx)
    phi = (2 * k + a + b) * (2 * k + a + b - 1) / (2 * k * (k + a + b))
    phi_p = (2 * k + a + b - 1) * (a ** 2 - b ** 2) / (
        2 * k * (k + a + b) * (2 * k + a + b - 2))
    phi_pp = (k + a - 1) * (k + b - 1) * (2 * k + a + b) / (
        k * (k + a + b) * (2 * k + a + b - 2))
    return phi, phi_p, -phi_pp


def kernel(x, edge_index, W_mlp, b_mlp, W_filters, b_filters, W_cls, b_cls):
    num_edges = edge_index.shape[1]
    pad = E_PAD - num_edges
    ar = jnp.arange(pad, dtype=jnp.int32)
    row_pad = N_NODES + (ar % (N_ACC - N_NODES))
    col_pad = (ar * 997) % N_NODES
    rowp = jnp.concatenate([edge_index[0], row_pad]).reshape(E_PAD // 128, 128)
    colp = jnp.concatenate([edge_index[1], col_pad]).reshape(E_PAD // 128, 128)

    zeros_acc = jnp.zeros((N_ACC, FDIM), jnp.float32)

    degp = _sc_degree(rowp, jnp.ones((128, FDIM), jnp.float32), zeros_acc)
    degs = degp[:, :, 0:8]
    h, ht, h0, q0 = _tc_mlp(x, W_mlp, b_mlp, degs,
                            W_filters[0], b_filters[0].reshape(1, FDIM))

    hs, qs = [h0], [q0]
    z, zp = h, h
    for k_idx in range(1, NK + 1):
        c0, c1, c2 = _jacobi_coefs(k_idx)
        p = _sc_spmm(colp, rowp, ht, zeros_acc)
        z_next, ht, hk, qk = _tc_combine(
            p, z, zp, degs, W_filters[k_idx],
            b_filters[k_idx].reshape(1, FDIM), c0, c1, c2)
        zp, z = z, z_next
        hs.append(hk)
        qs.append(qk)

    qsum = jnp.concatenate(qs, axis=0)
    out, z_tilde = _tc_attn(hs, qsum, W_cls, b_cls, W_cls.shape[1])
    return (out, z_tilde)


# speedup vs baseline: 1.0078x; 1.0078x over previous
import functools

import jax
import jax.numpy as jnp
from jax import lax
from jax.experimental import pallas as pl
from jax.experimental.pallas import tpu as pltpu
from jax.experimental.pallas import tpu_sc as plsc

N_NODES = 10000
FDIM = 128
NK = 10
JA = 1.0
JB = 1.0

NCHUNK = 80
EDGES_PER_TILE = NCHUNK * 128
E_PAD = 32 * EDGES_PER_TILE
N_ACC = 10240
ROWS_PER_TILE = N_ACC // 16

_sc_mesh = plsc.VectorSubcoreMesh(core_axis_name="c", subcore_axis_name="s")


HALF = NCHUNK // 2


@functools.partial(
    pl.kernel,
    mesh=_sc_mesh,
    out_type=jax.ShapeDtypeStruct((2, N_ACC, FDIM), jnp.float32),
    scratch_types=[
        pltpu.VMEM((HALF, 128), jnp.int32),
        pltpu.VMEM((HALF, 128), jnp.int32),
        pltpu.VMEM((2, 128, FDIM), jnp.float32),
        pltpu.VMEM_SHARED((N_ACC, FDIM), jnp.float32),
        pltpu.SemaphoreType.DMA,
        pltpu.SemaphoreType.DMA,
        pltpu.SemaphoreType.DMA,
        pltpu.SemaphoreType.DMA,
    ],
)
def _sc_spmm(col_hbm, row_hbm, ht_hbm, zeros_hbm, p_hbm,
             col_v, row_v, ring, acc, g0, g1, s0, s1):
    c = lax.axis_index("c")
    s = lax.axis_index("s")
    tile = c * 16 + s
    gsem = (g0, g1)
    ssem = (s0, s1)
    pltpu.sync_copy(zeros_hbm.at[pl.ds(s * ROWS_PER_TILE, ROWS_PER_TILE)],
                    acc.at[pl.ds(s * ROWS_PER_TILE, ROWS_PER_TILE)])
    plsc.subcore_barrier()

    for hf in range(2):
        base = tile * NCHUNK + hf * HALF
        pltpu.sync_copy(col_hbm.at[pl.ds(base, HALF)], col_v)
        pltpu.sync_copy(row_hbm.at[pl.ds(base, HALF)], row_v)
        pltpu.async_copy(ht_hbm.at[col_v.at[0]], ring.at[0], gsem[0])
        for j in range(HALF):
            b = j % 2
            pltpu.make_async_copy(ht_hbm.at[col_v.at[j]], ring.at[b],
                                  gsem[b]).wait()
            if j >= 1:
                pltpu.make_async_copy(ring.at[1 - b], acc.at[row_v.at[0]],
                                      ssem[1 - b]).wait()
            if j + 1 < HALF:
                pltpu.async_copy(ht_hbm.at[col_v.at[j + 1]], ring.at[1 - b],
                                 gsem[1 - b])
            pltpu.async_copy(ring.at[b], acc.at[row_v.at[j]], ssem[b], add=True)
        pltpu.make_async_copy(ring.at[(HALF - 1) % 2], acc.at[row_v.at[0]],
                              ssem[(HALF - 1) % 2]).wait()
    plsc.subcore_barrier()
    pltpu.sync_copy(acc.at[pl.ds(s * ROWS_PER_TILE, ROWS_PER_TILE)],
                    p_hbm.at[c].at[pl.ds(s * ROWS_PER_TILE, ROWS_PER_TILE)])


@functools.partial(
    pl.kernel,
    mesh=_sc_mesh,
    out_type=jax.ShapeDtypeStruct((2, N_ACC, FDIM), jnp.float32),
    scratch_types=[
        pltpu.VMEM((NCHUNK, 128), jnp.int32),
        pltpu.VMEM((128, FDIM), jnp.float32),
        pltpu.VMEM_SHARED((N_ACC, FDIM), jnp.float32),
        pltpu.SemaphoreType.DMA,
    ],
)
def _sc_degree(row_hbm, ones_hbm, zeros_hbm, p_hbm, row_v, ones_v, acc, sem):
    c = lax.axis_index("c")
    s = lax.axis_index("s")
    tile = c * 16 + s
    pltpu.sync_copy(row_hbm.at[pl.ds(tile * NCHUNK, NCHUNK)], row_v)
    pltpu.sync_copy(ones_hbm, ones_v)
    pltpu.sync_copy(zeros_hbm.at[pl.ds(s * ROWS_PER_TILE, ROWS_PER_TILE)],
                    acc.at[pl.ds(s * ROWS_PER_TILE, ROWS_PER_TILE)])
    plsc.subcore_barrier()
    for grp in range(NCHUNK // 16):
        for j in range(16):
            pltpu.async_copy(ones_v, acc.at[row_v.at[grp * 16 + j]], sem,
                             add=True)
        for j in range(16):
            pltpu.make_async_copy(ones_v, acc.at[row_v.at[0]], sem).wait()
    plsc.subcore_barrier()
    pltpu.sync_copy(acc.at[pl.ds(s * ROWS_PER_TILE, ROWS_PER_TILE)],
                    p_hbm.at[c].at[pl.ds(s * ROWS_PER_TILE, ROWS_PER_TILE)])


BLK = 400


def _mlp_body(x_ref, w_ref, b_ref, deg_ref, wf_ref, bf_ref,
              h_ref, ht_ref, hk_ref, q_ref):
    i = pl.program_id(0)
    h = jnp.maximum(jnp.dot(x_ref[...], w_ref[...],
                            preferred_element_type=jnp.float32) + b_ref[...], 0.0)
    deg = deg_ref[0, :, 0:1] + deg_ref[1, :, 0:1] + 1.0
    dis = lax.rsqrt(deg)
    h_ref[...] = h
    ht_ref[...] = h * dis
    hk = jnp.dot(h, wf_ref[...],
                 preferred_element_type=jnp.float32) + bf_ref[...]
    hk_ref[...] = hk
    part = jnp.sum(hk, axis=0, keepdims=True)

    @pl.when(i == 0)
    def _():
        q_ref[...] = part

    @pl.when(i != 0)
    def _():
        q_ref[...] += part


def _tc_mlp(x, w, b, degs, wf0, bf0):
    return pl.pallas_call(
        _mlp_body,
        grid=(N_NODES // BLK,),
        in_specs=[
            pl.BlockSpec((BLK, FDIM), lambda i: (i, 0)),
            pl.BlockSpec((FDIM, FDIM), lambda i: (0, 0)),
            pl.BlockSpec((FDIM,), lambda i: (0,)),
            pl.BlockSpec((2, BLK, 8), lambda i: (0, i, 0)),
            pl.BlockSpec((FDIM, FDIM), lambda i: (0, 0)),
            pl.BlockSpec((1, FDIM), lambda i: (0, 0)),
        ],
        out_specs=[
            pl.BlockSpec((BLK, FDIM), lambda i: (i, 0)),
            pl.BlockSpec((BLK, FDIM), lambda i: (i, 0)),
            pl.BlockSpec((BLK, FDIM), lambda i: (i, 0)),
            pl.BlockSpec((1, FDIM), lambda i: (0, 0)),
        ],
        out_shape=[
            jax.ShapeDtypeStruct((N_NODES, FDIM), jnp.float32),
            jax.ShapeDtypeStruct((N_NODES, FDIM), jnp.float32),
            jax.ShapeDtypeStruct((N_NODES, FDIM), jnp.float32),
            jax.ShapeDtypeStruct((1, FDIM), jnp.float32),
        ],
    )(x, w, b, degs, wf0, bf0)


def _combine_body(p_ref, z_ref, zp_ref, deg_ref, wf_ref, bf_ref,
                  zn_ref, htn_ref, hk_ref, q_ref, *, c0, c1, c2):
    i = pl.program_id(0)
    deg = deg_ref[0, :, 0:1] + deg_ref[1, :, 0:1] + 1.0
    dis = lax.rsqrt(deg)
    z = z_ref[...]
    sz = dis * (p_ref[0] + p_ref[1]) + (dis * dis) * z
    zn = c0 * sz + c1 * z + c2 * zp_ref[...]
    zn_ref[...] = zn
    htn_ref[...] = zn * dis
    hk = jnp.dot(zn, wf_ref[...],
                 preferred_element_type=jnp.float32) + bf_ref[...]
    hk_ref[...] = hk
    part = jnp.sum(hk, axis=0, keepdims=True)

    @pl.when(i == 0)
    def _():
        q_ref[...] = part

    @pl.when(i != 0)
    def _():
        q_ref[...] += part


def _tc_combine(p, z, zp, degs, wf, bf, c0, c1, c2):
    body = functools.partial(_combine_body, c0=c0, c1=c1, c2=c2)
    return pl.pallas_call(
        body,
        grid=(N_NODES // BLK,),
        in_specs=[
            pl.BlockSpec((2, BLK, FDIM), lambda i: (0, i, 0)),
            pl.BlockSpec((BLK, FDIM), lambda i: (i, 0)),
            pl.BlockSpec((BLK, FDIM), lambda i: (i, 0)),
            pl.BlockSpec((2, BLK, 8), lambda i: (0, i, 0)),
            pl.BlockSpec((FDIM, FDIM), lambda i: (0, 0)),
            pl.BlockSpec((1, FDIM), lambda i: (0, 0)),
        ],
        out_specs=[
            pl.BlockSpec((BLK, FDIM), lambda i: (i, 0)),
            pl.BlockSpec((BLK, FDIM), lambda i: (i, 0)),
            pl.BlockSpec((BLK, FDIM), lambda i: (i, 0)),
            pl.BlockSpec((1, FDIM), lambda i: (0, 0)),
        ],
        out_shape=[
            jax.ShapeDtypeStruct((N_NODES, FDIM), jnp.float32),
            jax.ShapeDtypeStruct((N_NODES, FDIM), jnp.float32),
            jax.ShapeDtypeStruct((N_NODES, FDIM), jnp.float32),
            jax.ShapeDtypeStruct((1, FDIM), jnp.float32),
        ],
    )(p, z, zp, degs, wf, bf)


def _attn_body(*refs):
    h_refs = refs[:NK + 1]
    qsum_ref, wc_ref, bc_ref, out_ref, zt_ref = refs[NK + 1:]
    q = qsum_ref[...] * (1.0 / N_NODES)
    cols = []
    for k in range(NK + 1):
        hk = h_refs[k][...]
        sk = jnp.sum(hk * q[k][None, :], axis=1, keepdims=True)
        cols.append(jnp.tanh(sk))
    scores = jnp.concatenate(cols, axis=1)
    m = jnp.max(scores, axis=1, keepdims=True)
    e = jnp.exp(scores - m)
    alpha = e / jnp.sum(e, axis=1, keepdims=True)
    zt = jnp.zeros_like(h_refs[0][...])
    for k in range(NK + 1):
        zt = zt + alpha[:, k:k + 1] * h_refs[k][...]
    zt = jnp.maximum(zt, 0.0)
    zt_ref[...] = zt
    out_ref[...] = jnp.dot(zt, wc_ref[...],
                           preferred_element_type=jnp.float32) + bc_ref[...]


def _tc_attn(hs, qsum, w_cls, b_cls, nout):
    hspec = pl.BlockSpec((BLK, FDIM), lambda i: (i, 0))
    return pl.pallas_call(
        _attn_body,
        grid=(N_NODES // BLK,),
        in_specs=[hspec] * (NK + 1) + [
            pl.BlockSpec((NK + 1, FDIM), lambda i: (0, 0)),
            pl.BlockSpec((FDIM, nout), lambda i: (0, 0)),
            pl.BlockSpec((nout,), lambda i: (0,)),
        ],
        out_specs=[
            pl.BlockSpec((BLK, nout), lambda i: (i, 0)),
            pl.BlockSpec((BLK, FDIM), lambda i: (i, 0)),
        ],
        out_shape=[
            jax.ShapeDtypeStruct((N_NODES, nout), jnp.float32),
            jax.ShapeDtypeStruct((N_NODES, FDIM), jnp.float32),
        ],
    )(*hs, qsum, w_cls, b_cls)


def _jacobi_coefs(k_idx):
    a, b = JA, JB
    if k_idx == 1:
        return (a + b + 2.0) / 2.0, (a - b) / 2.0, 0.0
    k = float(k_idx)
    phi = (2 * k + a + b) * (2 * k + a + b - 1) / (2 * k * (k + a + b))
    phi_p = (2 * k + a + b - 1) * (a ** 2 - b ** 2) / (
        2 * k * (k + a + b) * (2 * k + a + b - 2))
    phi_pp = (k + a - 1) * (k + b - 1) * (2 * k + a + b) / (
        k * (k + a + b) * (2 * k + a + b - 2))
    return phi, phi_p, -phi_pp


def kernel(x, edge_index, W_mlp, b_mlp, W_filters, b_filters, W_cls, b_cls):
    num_edges = edge_index.shape[1]
    pad = E_PAD - num_edges
    ar = jnp.arange(pad, dtype=jnp.int32)
    row_pad = N_NODES + (ar % (N_ACC - N_NODES))
    col_pad = (ar * 997) % N_NODES
    rowp = jnp.concatenate([edge_index[0], row_pad]).reshape(E_PAD // 128, 128)
    colp = jnp.concatenate([edge_index[1], col_pad]).reshape(E_PAD // 128, 128)

    zeros_acc = jnp.zeros((N_ACC, FDIM), jnp.float32)

    degp = _sc_degree(rowp, jnp.ones((128, FDIM), jnp.float32), zeros_acc)
    degs = degp[:, :, 0:8]
    h, ht, h0, q0 = _tc_mlp(x, W_mlp, b_mlp, degs,
                            W_filters[0], b_filters[0].reshape(1, FDIM))

    hs, qs = [h0], [q0]
    z, zp = h, h
    for k_idx in range(1, NK + 1):
        c0, c1, c2 = _jacobi_coefs(k_idx)
        p = _sc_spmm(colp, rowp, ht, zeros_acc)
        z_next, ht, hk, qk = _tc_combine(
            p, z, zp, degs, W_filters[k_idx],
            b_filters[k_idx].reshape(1, FDIM), c0, c1, c2)
        zp, z = z, z_next
        hs.append(hk)
        qs.append(qk)

    qsum = jnp.concatenate(qs, axis=0)
    out, z_tilde = _tc_attn(hs, qsum, W_cls, b_cls, W_cls.shape[1])
    return (out, z_tilde)
